# Initial kernel scaffold; baseline (speedup 1.0000x reference)
import jax
import jax.numpy as jnp
from jax.experimental import pallas as pl


def kernel(exp, x, edge_index, batch, params):
    raise NotImplementedError("write your pallas kernel here")


# speedup vs baseline: 1.1424x; 1.1424x over previous
import functools

import jax
import jax.numpy as jnp
from jax import lax
from jax.experimental import pallas as pl
from jax.experimental.pallas import tpu as pltpu
from jax.experimental.pallas import tpu_sc as plsc

N_NODES = 10000
N_EDGES = 320000
N_GRAPHS = 128
MAX_ATOM = 120
NUM_GENES = 954
DIM = 128

NC = 2
NS = 16
NW = NC * NS
EK = 128
N_CHUNKS = N_EDGES // EK
CHUNKS_PER_W = -(-N_CHUNKS // NW)
ROWS_PER_SUB = 624
TAIL_ROWS = N_NODES - NS * ROWS_PER_SUB

_PREC = jax.lax.Precision.HIGHEST

_sc_mesh = plsc.VectorSubcoreMesh(
    core_axis_name="c", subcore_axis_name="s", num_cores=NC, num_subcores=NS)


@functools.partial(
    pl.kernel,
    out_type=jax.ShapeDtypeStruct((NC * N_NODES, DIM), jnp.float32),
    mesh=_sc_mesh,
    scratch_types=[
        pltpu.VMEM((EK,), jnp.int32),
        pltpu.VMEM((EK,), jnp.int32),
        pltpu.VMEM((EK, DIM), jnp.float32),
        pltpu.VMEM_SHARED((N_NODES, DIM), jnp.float32),
        pltpu.SemaphoreType.DMA,
    ],
)
def _edge_agg(h_hbm, src_hbm, dst_hbm, out_hbm, src_v, dst_v, rows_v,
              acc_sh, sem):
    c = lax.axis_index("c")
    s = lax.axis_index("s")
    w = s * NC + c
    r0 = s * ROWS_PER_SUB
    pltpu.sync_copy(h_hbm.at[pl.ds(r0, ROWS_PER_SUB)],
                    acc_sh.at[pl.ds(r0, ROWS_PER_SUB)])

    @pl.when(s == NS - 1)
    def _():
        t0 = NS * ROWS_PER_SUB
        pltpu.sync_copy(h_hbm.at[pl.ds(t0, TAIL_ROWS)],
                        acc_sh.at[pl.ds(t0, TAIL_ROWS)])

    plsc.subcore_barrier()

    def body(j, carry):
        chunk = w + j * NW

        @pl.when(chunk < N_CHUNKS)
        def _():
            base = chunk * EK
            pltpu.sync_copy(src_hbm.at[pl.ds(base, EK)], src_v)
            pltpu.sync_copy(dst_hbm.at[pl.ds(base, EK)], dst_v)
            pltpu.async_copy(h_hbm.at[src_v], rows_v, sem).wait()
            pltpu.sync_copy(rows_v, acc_sh.at[dst_v], add=True)

        return carry

    lax.fori_loop(0, CHUNKS_PER_W, body, 0)
    plsc.subcore_barrier()
    pltpu.sync_copy(acc_sh.at[pl.ds(r0, ROWS_PER_SUB)],
                    out_hbm.at[pl.ds(c * N_NODES + r0, ROWS_PER_SUB)])

    @pl.when(s == NS - 1)
    def _():
        t0 = NS * ROWS_PER_SUB
        pltpu.sync_copy(acc_sh.at[pl.ds(t0, TAIL_ROWS)],
                        out_hbm.at[pl.ds(c * N_NODES + t0, TAIL_ROWS)])


IDS_PER_W = N_NODES // NW
IDS_TAIL = N_NODES - NW * IDS_PER_W


@functools.partial(
    pl.kernel,
    out_type=jax.ShapeDtypeStruct((N_NODES, DIM), jnp.float32),
    mesh=_sc_mesh,
    scratch_types=[
        pltpu.VMEM((IDS_PER_W + IDS_TAIL,), jnp.int32),
        pltpu.VMEM((IDS_PER_W + IDS_TAIL, DIM), jnp.float32),
        pltpu.SemaphoreType.DMA,
    ],
)
def _sc_embed(emb_hbm, x_hbm, out_hbm, ids_v, rows_v, sem):
    c = lax.axis_index("c")
    s = lax.axis_index("s")
    w = s * NC + c
    base = w * IDS_PER_W
    pltpu.sync_copy(x_hbm.at[pl.ds(base, IDS_PER_W)],
                    ids_v.at[pl.ds(0, IDS_PER_W)])
    pltpu.async_copy(emb_hbm.at[ids_v.at[pl.ds(0, IDS_PER_W)]],
                     rows_v.at[pl.ds(0, IDS_PER_W)], sem).wait()
    pltpu.sync_copy(rows_v.at[pl.ds(0, IDS_PER_W)],
                    out_hbm.at[pl.ds(base, IDS_PER_W)])

    @pl.when(w == NW - 1)
    def _():
        t0 = NW * IDS_PER_W
        pltpu.sync_copy(x_hbm.at[pl.ds(t0, IDS_TAIL)],
                        ids_v.at[pl.ds(0, IDS_TAIL)])
        pltpu.async_copy(emb_hbm.at[ids_v.at[pl.ds(0, IDS_TAIL)]],
                         rows_v.at[pl.ds(0, IDS_TAIL)], sem).wait()
        pltpu.sync_copy(rows_v.at[pl.ds(0, IDS_TAIL)],
                        out_hbm.at[pl.ds(t0, IDS_TAIL)])


def _mm_f32(a, b):
    return jax.lax.dot_general(
        a, b, (((1,), (0,)), ((), ())), precision=_PREC,
        preferred_element_type=jnp.float32)


def _mlp_bn(h, p0, p1, w1, b1, w2, b2, g, bb):
    z = p0 + p1 - h
    z1 = jnp.maximum(_mm_f32(z, w1) + b1, 0.0)
    h2 = _mm_f32(z1, w2) + b2
    hr = jnp.maximum(h2, 0.0)
    mu = jnp.mean(hr, axis=0, keepdims=True)
    var = jnp.mean((hr - mu) ** 2, axis=0, keepdims=True)
    return (hr - mu) / jnp.sqrt(var + 1e-5) * g + bb


def _dense_pool_body(h_ref, parts_ref, w1_ref, b1_ref, w2_ref, b2_ref,
                     g_ref, bb_ref, batch_ref, out_ref):
    hbn = _mlp_bn(
        h_ref[...], parts_ref[0], parts_ref[1], w1_ref[...], b1_ref[...],
        w2_ref[...], b2_ref[...], g_ref[...], bb_ref[...])
    gids = lax.broadcasted_iota(jnp.int32, (N_NODES, N_GRAPHS), 1)
    oh = (batch_ref[...] == gids).astype(jnp.float32)
    out_ref[...] = jax.lax.dot_general(
        oh, hbn, (((0,), (0,)), ((), ())),
        precision=_PREC, preferred_element_type=jnp.float32)


def _tc_dense_pool(h, parts, w1, b1, w2, b2, g, bb, batch2d):
    return pl.pallas_call(
        _dense_pool_body,
        out_shape=jax.ShapeDtypeStruct((N_GRAPHS, DIM), jnp.float32),
    )(h, parts, w1, b1, w2, b2, g, bb, batch2d)


def _head_body(pooled_ref, exp_ref, fxw_ref, fxb_ref, c1w_ref, c1b_ref,
               c2w_ref, c2b_ref, f1w_ref, f1b_ref, f2w_ref, f2b_ref,
               f3w_ref, f3b_ref, out_ref):
    mm = _mm_f32

    hg = jnp.maximum(mm(pooled_ref[...], fxw_ref[...]) + fxb_ref[...], 0.0)
    t = mm(exp_ref[...], c1w_ref[...]) + c1b_ref[...]
    cl = t * 0.5 * (1.0 + lax.erf(t * (2.0 ** -0.5)))
    cl = mm(cl, c2w_ref[...]) + c2b_ref[...]
    comb = jnp.concatenate([hg, cl], axis=1)
    o = jnp.maximum(mm(comb, f1w_ref[...]) + f1b_ref[...], 0.0)
    o = jnp.maximum(mm(o, f2w_ref[...]) + f2b_ref[...], 0.0)
    out_ref[...] = mm(o, f3w_ref[...]) + f3b_ref[...]


def _tc_head(pooled, exp, p):
    return pl.pallas_call(
        _head_body,
        out_shape=jax.ShapeDtypeStruct((N_GRAPHS, 1), jnp.float32),
    )(pooled, exp,
      p['fc_xd_W'], p['fc_xd_b'].reshape(1, -1),
      p['cl1_W'], p['cl1_b'].reshape(1, -1),
      p['cl2_W'], p['cl2_b'].reshape(1, -1),
      p['fc1_W'], p['fc1_b'].reshape(1, -1),
      p['fc2_W'], p['fc2_b'].reshape(1, -1),
      p['fc3_W'], p['fc3_b'].reshape(1, -1))


def kernel(exp, x, edge_index, batch, params):
    x1d = x.astype(jnp.int32)
    src = edge_index[0].astype(jnp.int32)
    dst = edge_index[1].astype(jnp.int32)
    batch2d = batch.astype(jnp.int32).reshape(N_NODES, 1)

    h = _sc_embed(params['emb'], x1d)
    convs = params['convs']

    for p in convs[:-1]:
        agg = jax.ops.segment_sum(h[src], dst, num_segments=N_NODES)
        z = h + agg
        z1 = jnp.maximum(z @ p['W1'] + p['b1'], 0.0)
        h2 = z1 @ p['W2'] + p['b2']
        hr = jnp.maximum(h2, 0.0)
        mu = jnp.mean(hr, axis=0)
        var = jnp.mean((hr - mu) ** 2, axis=0)
        h = (hr - mu) / jnp.sqrt(var + 1e-5) * p['bn_g'] + p['bn_b']

    p = convs[-1]
    parts = _edge_agg(h, src, dst).reshape(NC, N_NODES, DIM)
    pooled = _tc_dense_pool(
        h, parts, p['W1'], p['b1'].reshape(1, -1), p['W2'],
        p['b2'].reshape(1, -1), p['bn_g'].reshape(1, -1),
        p['bn_b'].reshape(1, -1), batch2d)

    return _tc_head(pooled, exp, params)


# speedup vs baseline: 1.4010x; 1.2263x over previous
import functools

import jax
import jax.numpy as jnp
from jax import lax
from jax.experimental import pallas as pl
from jax.experimental.pallas import tpu as pltpu
from jax.experimental.pallas import tpu_sc as plsc

N_NODES = 10000
N_EDGES = 320000
N_GRAPHS = 128
MAX_ATOM = 120
NUM_GENES = 954
DIM = 128

NC = 2
NS = 16
NW = NC * NS
EK = 128
N_CHUNKS = N_EDGES // EK
CHUNKS_PER_W = -(-N_CHUNKS // NW)
ROWS_PER_SUB = 624
TAIL_ROWS = N_NODES - NS * ROWS_PER_SUB

_PREC = jax.lax.Precision.HIGHEST

_sc_mesh = plsc.VectorSubcoreMesh(
    core_axis_name="c", subcore_axis_name="s", num_cores=NC, num_subcores=NS)


@functools.partial(
    pl.kernel,
    out_type=jax.ShapeDtypeStruct((NC * N_NODES, DIM), jnp.float32),
    mesh=_sc_mesh,
    scratch_types=[
        pltpu.VMEM((EK,), jnp.int32),
        pltpu.VMEM((EK,), jnp.int32),
        pltpu.VMEM((EK, DIM), jnp.float32),
        pltpu.VMEM_SHARED((N_NODES, DIM), jnp.float32),
        pltpu.SemaphoreType.DMA,
    ],
)
def _edge_agg(h_hbm, src_hbm, dst_hbm, out_hbm, src_v, dst_v, rows_v,
              acc_sh, sem):
    c = lax.axis_index("c")
    s = lax.axis_index("s")
    w = s * NC + c
    r0 = s * ROWS_PER_SUB
    pltpu.sync_copy(h_hbm.at[pl.ds(r0, ROWS_PER_SUB)],
                    acc_sh.at[pl.ds(r0, ROWS_PER_SUB)])

    @pl.when(s == NS - 1)
    def _():
        t0 = NS * ROWS_PER_SUB
        pltpu.sync_copy(h_hbm.at[pl.ds(t0, TAIL_ROWS)],
                        acc_sh.at[pl.ds(t0, TAIL_ROWS)])

    plsc.subcore_barrier()

    def body(j, carry):
        chunk = w + j * NW

        @pl.when(chunk < N_CHUNKS)
        def _():
            base = chunk * EK
            pltpu.sync_copy(src_hbm.at[pl.ds(base, EK)], src_v)
            pltpu.sync_copy(dst_hbm.at[pl.ds(base, EK)], dst_v)
            pltpu.async_copy(h_hbm.at[src_v], rows_v, sem).wait()
            pltpu.sync_copy(rows_v, acc_sh.at[dst_v], add=True)

        return carry

    lax.fori_loop(0, CHUNKS_PER_W, body, 0)
    plsc.subcore_barrier()
    pltpu.sync_copy(acc_sh.at[pl.ds(r0, ROWS_PER_SUB)],
                    out_hbm.at[pl.ds(c * N_NODES + r0, ROWS_PER_SUB)])

    @pl.when(s == NS - 1)
    def _():
        t0 = NS * ROWS_PER_SUB
        pltpu.sync_copy(acc_sh.at[pl.ds(t0, TAIL_ROWS)],
                        out_hbm.at[pl.ds(c * N_NODES + t0, TAIL_ROWS)])


IDS_PER_W = N_NODES // NW
IDS_TAIL = N_NODES - NW * IDS_PER_W


@functools.partial(
    pl.kernel,
    out_type=jax.ShapeDtypeStruct((N_NODES, DIM), jnp.float32),
    mesh=_sc_mesh,
    scratch_types=[
        pltpu.VMEM((IDS_PER_W + IDS_TAIL,), jnp.int32),
        pltpu.VMEM((IDS_PER_W + IDS_TAIL, DIM), jnp.float32),
        pltpu.SemaphoreType.DMA,
    ],
)
def _sc_embed(emb_hbm, x_hbm, out_hbm, ids_v, rows_v, sem):
    c = lax.axis_index("c")
    s = lax.axis_index("s")
    w = s * NC + c
    base = w * IDS_PER_W
    pltpu.sync_copy(x_hbm.at[pl.ds(base, IDS_PER_W)],
                    ids_v.at[pl.ds(0, IDS_PER_W)])
    pltpu.async_copy(emb_hbm.at[ids_v.at[pl.ds(0, IDS_PER_W)]],
                     rows_v.at[pl.ds(0, IDS_PER_W)], sem).wait()
    pltpu.sync_copy(rows_v.at[pl.ds(0, IDS_PER_W)],
                    out_hbm.at[pl.ds(base, IDS_PER_W)])

    @pl.when(w == NW - 1)
    def _():
        t0 = NW * IDS_PER_W
        pltpu.sync_copy(x_hbm.at[pl.ds(t0, IDS_TAIL)],
                        ids_v.at[pl.ds(0, IDS_TAIL)])
        pltpu.async_copy(emb_hbm.at[ids_v.at[pl.ds(0, IDS_TAIL)]],
                         rows_v.at[pl.ds(0, IDS_TAIL)], sem).wait()
        pltpu.sync_copy(rows_v.at[pl.ds(0, IDS_TAIL)],
                        out_hbm.at[pl.ds(t0, IDS_TAIL)])


def _mm_f32(a, b):
    return jax.lax.dot_general(
        a, b, (((1,), (0,)), ((), ())), precision=_PREC,
        preferred_element_type=jnp.float32)


def _mlp_bn(h, p0, p1, w1, b1, w2, b2, g, bb):
    z = p0 + p1 - h
    z1 = jnp.maximum(_mm_f32(z, w1) + b1, 0.0)
    h2 = _mm_f32(z1, w2) + b2
    hr = jnp.maximum(h2, 0.0)
    mu = jnp.mean(hr, axis=0, keepdims=True)
    var = jnp.mean((hr - mu) ** 2, axis=0, keepdims=True)
    return (hr - mu) / jnp.sqrt(var + 1e-5) * g + bb


def _dense_pool_body(h_ref, parts_ref, w1_ref, b1_ref, w2_ref, b2_ref,
                     g_ref, bb_ref, batch_ref, out_ref):
    hbn = _mlp_bn(
        h_ref[...], parts_ref[0], parts_ref[1], w1_ref[...], b1_ref[...],
        w2_ref[...], b2_ref[...], g_ref[...], bb_ref[...])
    gids = lax.broadcasted_iota(jnp.int32, (N_NODES, N_GRAPHS), 1)
    oh = (batch_ref[...] == gids).astype(jnp.float32)
    out_ref[...] = jax.lax.dot_general(
        oh, hbn, (((0,), (0,)), ((), ())),
        precision=_PREC, preferred_element_type=jnp.float32)


def _tc_dense_pool(h, parts, w1, b1, w2, b2, g, bb, batch2d):
    return pl.pallas_call(
        _dense_pool_body,
        out_shape=jax.ShapeDtypeStruct((N_GRAPHS, DIM), jnp.float32),
    )(h, parts, w1, b1, w2, b2, g, bb, batch2d)


def _head_body(pooled_ref, exp_ref, fxw_ref, fxb_ref, c1w_ref, c1b_ref,
               c2w_ref, c2b_ref, f1w_ref, f1b_ref, f2w_ref, f2b_ref,
               f3w_ref, f3b_ref, out_ref):
    mm = _mm_f32

    hg = jnp.maximum(mm(pooled_ref[...], fxw_ref[...]) + fxb_ref[...], 0.0)
    t = mm(exp_ref[...], c1w_ref[...]) + c1b_ref[...]
    cl = t * 0.5 * (1.0 + lax.erf(t * (2.0 ** -0.5)))
    cl = mm(cl, c2w_ref[...]) + c2b_ref[...]
    comb = jnp.concatenate([hg, cl], axis=1)
    o = jnp.maximum(mm(comb, f1w_ref[...]) + f1b_ref[...], 0.0)
    o = jnp.maximum(mm(o, f2w_ref[...]) + f2b_ref[...], 0.0)
    out_ref[...] = mm(o, f3w_ref[...]) + f3b_ref[...]


def _tc_head(pooled, exp, p):
    return pl.pallas_call(
        _head_body,
        out_shape=jax.ShapeDtypeStruct((N_GRAPHS, 1), jnp.float32),
    )(pooled, exp,
      p['fc_xd_W'], p['fc_xd_b'].reshape(1, -1),
      p['cl1_W'], p['cl1_b'].reshape(1, -1),
      p['cl2_W'], p['cl2_b'].reshape(1, -1),
      p['fc1_W'], p['fc1_b'].reshape(1, -1),
      p['fc2_W'], p['fc2_b'].reshape(1, -1),
      p['fc3_W'], p['fc3_b'].reshape(1, -1))


def kernel(exp, x, edge_index, batch, params):
    x1d = x.astype(jnp.int32)
    src = edge_index[0].astype(jnp.int32)
    dst = edge_index[1].astype(jnp.int32)
    batch2d = batch.astype(jnp.int32).reshape(N_NODES, 1)

    h = _sc_embed(params['emb'], x1d)
    convs = params['convs']

    for i, p in enumerate(convs[:-1]):
        if i == 3:
            parts = _edge_agg(h, src, dst).reshape(NC, N_NODES, DIM)
            z = parts[0] + parts[1] - h
        else:
            agg = jax.ops.segment_sum(h[src], dst, num_segments=N_NODES)
            z = h + agg
        z1 = jnp.maximum(z @ p['W1'] + p['b1'], 0.0)
        h2 = z1 @ p['W2'] + p['b2']
        hr = jnp.maximum(h2, 0.0)
        mu = jnp.mean(hr, axis=0)
        var = jnp.mean((hr - mu) ** 2, axis=0)
        h = (hr - mu) / jnp.sqrt(var + 1e-5) * p['bn_g'] + p['bn_b']

    p = convs[-1]
    parts = _edge_agg(h, src, dst).reshape(NC, N_NODES, DIM)
    pooled = _tc_dense_pool(
        h, parts, p['W1'], p['b1'].reshape(1, -1), p['W2'],
        p['b2'].reshape(1, -1), p['bn_g'].reshape(1, -1),
        p['bn_b'].reshape(1, -1), batch2d)

    return _tc_head(pooled, exp, params)
